# Initial kernel scaffold; baseline (speedup 1.0000x reference)
import jax
import jax.numpy as jnp
from jax.experimental import pallas as pl


def kernel(x, card_emb_out, turn_table, pos_table, civ_table, face_table, action_table, coin_W, coin_b):
    raise NotImplementedError("write your pallas kernel here")


# speedup vs baseline: 6.2322x; 6.2322x over previous
import functools

import jax
import jax.numpy as jnp
from jax import lax
from jax.experimental import pallas as pl


def _total_emb_kernel(x_ref, card_ref, wlut_ref, coinw_ref, coinb_ref, out_ref, *, o, tile):
    x = x_ref[...]
    xi = x.astype(jnp.int32)
    a_turn = xi[:, 0:1]
    a_pos = xi[:, 3:4] + (o + 20)
    a_civ = xi[:, 4:5] + 50
    a_face = xi[:, 5:6] + 58
    a_act = xi[:, 2:3] + 61
    iota = lax.broadcasted_iota(jnp.int32, (tile, 65), 1)
    onehot = ((iota == a_turn).astype(jnp.float32)
              + (iota == a_pos).astype(jnp.float32)
              + (iota == a_civ).astype(jnp.float32)
              + (iota == a_face).astype(jnp.float32)
              + (iota == a_act).astype(jnp.float32))
    lut_sum = jnp.dot(onehot, wlut_ref[...], preferred_element_type=jnp.float32)
    coin = jnp.dot(x[:, 6:], coinw_ref[...], preferred_element_type=jnp.float32)
    out_ref[...] = lut_sum + coin + card_ref[...] + coinb_ref[...]


def kernel(x, card_emb_out, turn_table, pos_table, civ_table, face_table, action_table, coin_W, coin_b):
    B, S, F = x.shape
    D = card_emb_out.shape[-1]
    N = B * S

    n = (S - 6) // 19
    lookup = {3: 0, 4: 4, 5: 9, 6: 15, 7: 22}
    o = lookup.get(n, -100)

    w_lut = jnp.concatenate(
        [turn_table, pos_table, civ_table, face_table, action_table], axis=0)

    x2 = x.reshape(N, F)
    card2 = card_emb_out.reshape(N, D)
    coin_b2 = coin_b.reshape(1, D)

    tile = 2016
    grid = N // tile

    out = pl.pallas_call(
        functools.partial(_total_emb_kernel, o=o, tile=tile),
        grid=(grid,),
        in_specs=[
            pl.BlockSpec((tile, F), lambda i: (i, 0)),
            pl.BlockSpec((tile, D), lambda i: (i, 0)),
            pl.BlockSpec(w_lut.shape, lambda i: (0, 0)),
            pl.BlockSpec(coin_W.shape, lambda i: (0, 0)),
            pl.BlockSpec((1, D), lambda i: (0, 0)),
        ],
        out_specs=pl.BlockSpec((tile, D), lambda i: (i, 0)),
        out_shape=jax.ShapeDtypeStruct((N, D), jnp.float32),
    )(x2, card2, w_lut, coin_W, coin_b2)

    return out.reshape(B, S, D)


# speedup vs baseline: 7.6140x; 1.2217x over previous
import functools

import jax
import jax.numpy as jnp
from jax import lax
from jax.experimental import pallas as pl


def _total_emb_kernel(x_ref, card_ref, wlut_ref, coinw_ref, coinb_ref, out_ref, *, o, bb, S):
    x = x_ref[...]
    xi = x.astype(jnp.int32)
    a_turn = xi[:, :, 0:1]
    a_pos = xi[:, :, 3:4] + (o + 20)
    a_civ = xi[:, :, 4:5] + 50
    a_face = xi[:, :, 5:6] + 58
    a_act = xi[:, :, 2:3] + 61
    iota = lax.broadcasted_iota(jnp.int32, (bb, S, 65), 2)
    onehot = ((iota == a_turn).astype(jnp.float32)
              + (iota == a_pos).astype(jnp.float32)
              + (iota == a_civ).astype(jnp.float32)
              + (iota == a_face).astype(jnp.float32)
              + (iota == a_act).astype(jnp.float32))
    xc = x[:, :, 6:]
    wlut = wlut_ref[...]
    coinw = coinw_ref[...]
    coinb = coinb_ref[...]
    for i in range(bb):
        lut_sum = jnp.dot(onehot[i], wlut, preferred_element_type=jnp.float32)
        coin = jnp.dot(xc[i], coinw, preferred_element_type=jnp.float32)
        out_ref[i] = lut_sum + coin + card_ref[i] + coinb


def kernel(x, card_emb_out, turn_table, pos_table, civ_table, face_table, action_table, coin_W, coin_b):
    B, S, F = x.shape
    D = card_emb_out.shape[-1]

    n = (S - 6) // 19
    lookup = {3: 0, 4: 4, 5: 9, 6: 15, 7: 22}
    o = lookup.get(n, -100)

    w_lut = jnp.concatenate(
        [turn_table, pos_table, civ_table, face_table, action_table], axis=0)
    coin_b2 = coin_b.reshape(1, D)

    bb = 16
    grid = B // bb

    return pl.pallas_call(
        functools.partial(_total_emb_kernel, o=o, bb=bb, S=S),
        grid=(grid,),
        in_specs=[
            pl.BlockSpec((bb, S, F), lambda i: (i, 0, 0)),
            pl.BlockSpec((bb, S, D), lambda i: (i, 0, 0)),
            pl.BlockSpec(w_lut.shape, lambda i: (0, 0)),
            pl.BlockSpec(coin_W.shape, lambda i: (0, 0)),
            pl.BlockSpec((1, D), lambda i: (0, 0)),
        ],
        out_specs=pl.BlockSpec((bb, S, D), lambda i: (i, 0, 0)),
        out_shape=jax.ShapeDtypeStruct((B, S, D), jnp.float32),
    )(x, card_emb_out, w_lut, coin_W, coin_b2)


# speedup vs baseline: 8.4287x; 1.1070x over previous
import functools

import jax
import jax.numpy as jnp
from jax import lax
from jax.experimental import pallas as pl


def _total_emb_kernel(x_ref, card_ref, wlut_ref, coinw_ref, coinb_ref, out_ref, *, o, bb, S):
    x = x_ref[...]
    xi = x.astype(jnp.int32)
    a_turn = xi[:, :, 0:1]
    a_pos = xi[:, :, 3:4] + (o + 20)
    a_civ = xi[:, :, 4:5] + 50
    a_face = xi[:, :, 5:6] + 58
    a_act = xi[:, :, 2:3] + 61
    iota = lax.broadcasted_iota(jnp.int32, (bb, S, 65), 2)
    onehot = ((iota == a_turn).astype(jnp.float32)
              + (iota == a_pos).astype(jnp.float32)
              + (iota == a_civ).astype(jnp.float32)
              + (iota == a_face).astype(jnp.float32)
              + (iota == a_act).astype(jnp.float32))
    xc = x[:, :, 6:]
    wlut = wlut_ref[...]
    coinw = coinw_ref[...]
    coinb = coinb_ref[...]
    for i in range(bb):
        lut_sum = jnp.dot(onehot[i], wlut, preferred_element_type=jnp.float32)
        coin = jnp.dot(xc[i], coinw, preferred_element_type=jnp.float32)
        out_ref[i] = lut_sum + coin + card_ref[i] + coinb


def kernel(x, card_emb_out, turn_table, pos_table, civ_table, face_table, action_table, coin_W, coin_b):
    B, S, F = x.shape
    D = card_emb_out.shape[-1]

    n = (S - 6) // 19
    lookup = {3: 0, 4: 4, 5: 9, 6: 15, 7: 22}
    o = lookup.get(n, -100)

    w_lut = jnp.concatenate(
        [turn_table, pos_table, civ_table, face_table, action_table], axis=0)
    coin_b2 = coin_b.reshape(1, D)

    bb = 32
    grid = B // bb

    return pl.pallas_call(
        functools.partial(_total_emb_kernel, o=o, bb=bb, S=S),
        grid=(grid,),
        in_specs=[
            pl.BlockSpec((bb, S, F), lambda i: (i, 0, 0)),
            pl.BlockSpec((bb, S, D), lambda i: (i, 0, 0)),
            pl.BlockSpec(w_lut.shape, lambda i: (0, 0)),
            pl.BlockSpec(coin_W.shape, lambda i: (0, 0)),
            pl.BlockSpec((1, D), lambda i: (0, 0)),
        ],
        out_specs=pl.BlockSpec((bb, S, D), lambda i: (i, 0, 0)),
        out_shape=jax.ShapeDtypeStruct((B, S, D), jnp.float32),
    )(x, card_emb_out, w_lut, coin_W, coin_b2)


# speedup vs baseline: 8.6624x; 1.0277x over previous
import functools

import jax
import jax.numpy as jnp
from jax import lax
from jax.experimental import pallas as pl


def _total_emb_kernel(x_ref, card_ref, wlut_ref, coinw_ref, coinb_ref, out_ref, *, o, bb, S):
    x = x_ref[...]
    xi = x.astype(jnp.int32)
    a_turn = xi[:, :, 0:1]
    a_pos = xi[:, :, 3:4] + (o + 20)
    a_civ = xi[:, :, 4:5] + 50
    a_face = xi[:, :, 5:6] + 58
    a_act = xi[:, :, 2:3] + 61
    iota = lax.broadcasted_iota(jnp.int32, (bb, S, 65), 2)
    onehot = ((iota == a_turn).astype(jnp.float32)
              + (iota == a_pos).astype(jnp.float32)
              + (iota == a_civ).astype(jnp.float32)
              + (iota == a_face).astype(jnp.float32)
              + (iota == a_act).astype(jnp.float32))
    xc = x[:, :, 6:]
    wlut = wlut_ref[...]
    coinw = coinw_ref[...]
    coinb = coinb_ref[...]
    for i in range(bb):
        lut_sum = jnp.dot(onehot[i], wlut, preferred_element_type=jnp.float32)
        coin = jnp.dot(xc[i], coinw, preferred_element_type=jnp.float32)
        out_ref[i] = lut_sum + coin + card_ref[i] + coinb


def kernel(x, card_emb_out, turn_table, pos_table, civ_table, face_table, action_table, coin_W, coin_b):
    B, S, F = x.shape
    D = card_emb_out.shape[-1]

    n = (S - 6) // 19
    lookup = {3: 0, 4: 4, 5: 9, 6: 15, 7: 22}
    o = lookup.get(n, -100)

    w_lut = jnp.concatenate(
        [turn_table, pos_table, civ_table, face_table, action_table], axis=0)
    coin_b2 = coin_b.reshape(1, D)

    bb = 64
    grid = B // bb

    return pl.pallas_call(
        functools.partial(_total_emb_kernel, o=o, bb=bb, S=S),
        grid=(grid,),
        in_specs=[
            pl.BlockSpec((bb, S, F), lambda i: (i, 0, 0)),
            pl.BlockSpec((bb, S, D), lambda i: (i, 0, 0)),
            pl.BlockSpec(w_lut.shape, lambda i: (0, 0)),
            pl.BlockSpec(coin_W.shape, lambda i: (0, 0)),
            pl.BlockSpec((1, D), lambda i: (0, 0)),
        ],
        out_specs=pl.BlockSpec((bb, S, D), lambda i: (i, 0, 0)),
        out_shape=jax.ShapeDtypeStruct((B, S, D), jnp.float32),
    )(x, card_emb_out, w_lut, coin_W, coin_b2)
